# baseline (speedup 1.0000x reference)
import functools

import jax
import jax.numpy as jnp
from jax.experimental import pallas as pl
from jax.experimental.pallas import tpu as pltpu

_F32 = jnp.float32
_I32 = jnp.int32
def _mm(a, b):
    return jnp.dot(a, b, preferred_element_type=_F32)


def _lrelu(x):
    return jnp.where(x >= 0, x, 0.1 * x)


def _make_masks():
    lane = jax.lax.broadcasted_iota(_I32, (1, 1024), 1)
    h = lane // 32
    w = lane % 32
    masks = {}
    for dy in (-1, 0, 1):
        for dx in (-1, 0, 1):
            if dy == 0 and dx == 0:
                continue
            cond = ((h + dy >= 0) & (h + dy < 32)
                    & (w + dx >= 0) & (w + dx < 32))
            masks[(dy, dx)] = cond
    return masks


def _sh(x, dy, dx, masks):
    if dy == 0 and dx == 0:
        return x
    r = jnp.roll(x, -(32 * dy + dx), axis=1)
    return jnp.where(masks[(dy, dx)], r, 0.0)


def _conv3(x, w9, b, masks):
    acc = None
    for ky in range(3):
        for kx in range(3):
            t = _mm(w9[3 * ky + kx], _sh(x, ky - 1, kx - 1, masks))
            acc = t if acc is None else acc + t
    return acc + b


def _phase_conv(z, w9, b, r, masks):
    out = {}
    for a in range(r):
        for bph in range(r):
            acc = None
            for dy in (-1, 0, 1):
                for dx in (-1, 0, 1):
                    ia = (a + dy) % r
                    ib = (bph + dx) % r
                    sy = (a + dy) // r
                    sx = (bph + dx) // r
                    t = _mm(w9[3 * (dy + 1) + (dx + 1)],
                            _sh(z[(ia, ib)], sy, sx, masks))
                    acc = t if acc is None else acc + t
            out[(a, bph)] = acc + b
    return out


def _cls_body(xc_ref, w0, b0, w1, b1, w2, b2, w3, b3, w4, b4, fcw, fcb,
              route_ref):
    h = _lrelu(_mm(xc_ref[...], w0[...]) + b0[...])
    h = _lrelu(_mm(h, w1[...]) + b1[...])
    h = _lrelu(_mm(h, w2[...]) + b2[...])
    h = _lrelu(_mm(h, w3[...]) + b3[...])
    h = _mm(h, w4[...]) + b4[...]
    m = jnp.mean(h.reshape(64, 64, 32), axis=1)
    logits = _mm(m, fcw[...]) + fcb[...]
    l0 = logits[:, 0:1]
    l1 = logits[:, 1:2]
    l2 = logits[:, 2:3]
    a_col = jnp.where((l0 >= l1) & (l0 >= l2), 0,
                      jnp.where(l1 >= l2, 1, 2)).astype(_I32)
    a_row = a_col.reshape(1, 64)

    ir = jax.lax.broadcasted_iota(_I32, (64, 64), 0)
    ic = jax.lax.broadcasted_iota(_I32, (64, 64), 1)
    before = (a_row < a_col) | ((a_row == a_col) & (ic < ir))
    rank = jnp.sum(before.astype(_I32), axis=1, keepdims=True)
    rank_row = rank.reshape(1, 64)
    sel = (rank_row == ir).astype(_I32)
    s_col = jnp.sum(sel * ic, axis=1, keepdims=True)
    s_row = s_col.reshape(1, 64)

    e8 = jax.lax.broadcasted_iota(_I32, (8, 64), 0)
    cnt8 = jnp.sum((a_row == e8).astype(_I32), axis=1, keepdims=True)
    cnt_row = cnt8.reshape(1, 8)
    c0 = cnt_row[:, 0:1]
    c1 = cnt_row[:, 1:2]
    zero = jnp.zeros((1, 1), _I32)
    off_row = jnp.concatenate([zero, c0, c0 + c1], axis=1)
    cnt_p = jnp.pad(cnt_row, ((0, 0), (0, 56)))
    off_p = jnp.pad(off_row, ((0, 0), (0, 61)))
    zrows = jnp.zeros((4, 64), _I32)
    route_ref[...] = jnp.concatenate(
        [a_row, s_row, cnt_p, off_p, zrows], axis=0)


def _classify(x, cls):
    xc = x.reshape(64, 3, 8, 4, 8, 4).transpose(0, 2, 4, 1, 3, 5)
    xc = xc.reshape(4096, 48)
    args = [
        xc,
        cls['c0_w'].reshape(128, 48).T, cls['c0_b'].reshape(1, 128),
        cls['c1_w'].reshape(128, 128).T, cls['c1_b'].reshape(1, 128),
        cls['c2_w'].reshape(128, 128).T, cls['c2_b'].reshape(1, 128),
        cls['c3_w'].reshape(128, 128).T, cls['c3_b'].reshape(1, 128),
        cls['c4_w'].reshape(32, 128).T, cls['c4_b'].reshape(1, 32),
        cls['fc_w'].T, cls['fc_b'].reshape(1, 3),
    ]
    route = pl.pallas_call(
        _cls_body,
        out_shape=jax.ShapeDtypeStruct((8, 64), _I32),
    )(*args)
    return route


def _carn_weights(p, nf):
    q = nf // 4

    def tapify(w):
        return w.transpose(2, 3, 0, 1).reshape(9, w.shape[0], w.shape[1])

    out = [tapify(p['entry_w']), p['entry_b'].reshape(nf, 1)]
    for i in range(3):
        gw = p['b%d_gw' % i]
        full = jnp.zeros((nf, nf, 3, 3), _F32)
        for g in range(4):
            full = full.at[g * q:(g + 1) * q, g * q:(g + 1) * q].set(
                gw[g * q:(g + 1) * q])
        out += [tapify(full), p['b%d_gb' % i].reshape(nf, 1),
                p['b%d_pw' % i].reshape(nf, nf), p['b%d_pb' % i].reshape(nf, 1)]
    perm = jnp.arange(4 * nf).reshape(4, nf)
    perm = (jnp.arange(nf)[None, :] * 4 + jnp.arange(4)[:, None]).reshape(-1)
    for nm in ('up1', 'up2'):
        w9 = tapify(p[nm + '_w'])
        out += [w9[:, perm, :], p[nm + '_b'][perm].reshape(4 * nf, 1)]
    out += [tapify(p['exit_w']), p['exit_b'].reshape(3, 1)]
    return out


def _carn_body(nf, info_ref, x_ref, ew9, eb, b0w9, b0b, b0pw, b0pb,
               b1w9, b1b, b1pw, b1pb, b2w9, b2b, b2pw, b2pb,
               u1w9, u1b, u2w9, u2b, xw9, xb, out_ref):
    i = pl.program_id(0)

    @pl.when(i < info_ref[1])
    def _():
        masks = _make_masks()
        x = x_ref[0]
        h = jnp.maximum(_conv3(x, ew9, eb[...], masks), 0.0)
        for bw9, bb, bpw, bpb in ((b0w9, b0b, b0pw, b0pb),
                                  (b1w9, b1b, b1pw, b1pb),
                                  (b2w9, b2b, b2pw, b2pb)):
            t = jnp.maximum(_conv3(h, bw9, bb[...], masks), 0.0)
            t = _mm(bpw[...], t) + bpb[...]
            h = jnp.maximum(h + t, 0.0)
        u = jnp.maximum(_conv3(h, u1w9, u1b[...], masks), 0.0)
        z = {(a, b): u[(2 * a + b) * nf:(2 * a + b + 1) * nf]
             for a in range(2) for b in range(2)}
        v = _phase_conv(z, u2w9, u2b[...], 2, masks)
        p16 = {}
        for a in range(2):
            for b in range(2):
                vp = jnp.maximum(v[(a, b)], 0.0)
                for e in range(2):
                    for f in range(2):
                        k = 2 * e + f
                        p16[(2 * a + e, 2 * b + f)] = (
                            vp[k * nf:(k + 1) * nf])
        y = _phase_conv(p16, xw9, xb[...], 4, masks)
        for a in range(4):
            rows = jnp.stack(
                [y[(a, b)].reshape(3, 32, 32) for b in range(4)], axis=3)
            out_ref[0, :, :, a, :] = rows.reshape(3, 32, 128)


def _run_expert(xf, info, wlist, nf):
    def in_map(i, info_ref):
        start = info_ref[0]
        count = info_ref[1]
        idx = jnp.minimum(
            start + jnp.minimum(i, jnp.maximum(count - 1, 0)), 63)
        return (info_ref[2 + idx], 0, 0)

    def out_map(i, info_ref):
        start = info_ref[0]
        count = info_ref[1]
        idx = jnp.minimum(
            start + jnp.minimum(i, jnp.maximum(count - 1, 0)), 63)
        slot = jnp.where(i < count, info_ref[2 + idx], 64)
        return (slot, 0, 0, 0, 0)

    in_specs = [pl.BlockSpec((1, 3, 1024), in_map)]
    for w in wlist:
        in_specs.append(pl.BlockSpec(
            w.shape, functools.partial(
                lambda nd, i, info_ref: (0,) * nd, w.ndim)))

    grid_spec = pltpu.PrefetchScalarGridSpec(
        num_scalar_prefetch=1,
        grid=(64,),
        in_specs=in_specs,
        out_specs=pl.BlockSpec((1, 3, 32, 4, 128), out_map),
    )
    return pl.pallas_call(
        functools.partial(_carn_body, nf),
        grid_spec=grid_spec,
        out_shape=jax.ShapeDtypeStruct((65, 3, 32, 4, 128), _F32),
    )(info, xf, *wlist)


def _impl(x, params):
    route = _classify(x, params['cls'])
    assign = route[0]
    sorted_idx = route[1]
    counts = route[2, :3]
    offsets = route[3, :3]

    xf = x.reshape(64, 3, 1024)
    ys = []
    for e, (name, nf) in enumerate((('net1', 36), ('net2', 52), ('net3', 64))):
        info = jnp.concatenate(
            [offsets[e:e + 1], counts[e:e + 1], sorted_idx]).astype(_I32)
        wlist = _carn_weights(params[name], nf)
        ys.append(_run_expert(xf, info, wlist, nf))

    sel = assign.reshape(64, 1, 1, 1, 1)
    out = jnp.where(sel == 0, ys[0][:64],
                    jnp.where(sel == 1, ys[1][:64], ys[2][:64]))
    return out.reshape(64, 3, 128, 128), counts


_impl_jit = jax.jit(_impl)


def kernel(x, params):
    return _impl_jit(x, params)


# speedup vs baseline: 4.1088x; 4.1088x over previous
import functools

import jax
import jax.numpy as jnp
from jax.experimental import pallas as pl
from jax.experimental.pallas import tpu as pltpu

_F32 = jnp.float32
_I32 = jnp.int32
_BF16 = jnp.bfloat16


def _mm(a, b):
    return jnp.dot(a, b, preferred_element_type=_F32)


def _lrelu(x):
    return jnp.where(x >= 0, x, 0.1 * x)


def _make_masks():
    lane = jax.lax.broadcasted_iota(_I32, (1, 1024), 1)
    h = lane // 32
    w = lane % 32
    masks = {}
    for dy in (-1, 0, 1):
        for dx in (-1, 0, 1):
            if dy == 0 and dx == 0:
                continue
            cond = ((h + dy >= 0) & (h + dy < 32)
                    & (w + dx >= 0) & (w + dx < 32))
            masks[(dy, dx)] = cond
    return masks


def _sh(x, dy, dx, masks):
    if dy == 0 and dx == 0:
        return x
    r = jnp.roll(x, -(32 * dy + dx), axis=1)
    return jnp.where(masks[(dy, dx)], r, 0.0)


def _conv3(x, w9, b, masks):
    xb = x.astype(_BF16)
    acc = None
    for ky in range(3):
        for kx in range(3):
            t = _mm(w9[3 * ky + kx], _sh(xb, ky - 1, kx - 1, masks))
            acc = t if acc is None else acc + t
    return acc + b


def _phase_conv(z, w9, b, r, masks):
    zb = {k: v.astype(_BF16) for k, v in z.items()}
    out = {}
    for a in range(r):
        for bph in range(r):
            acc = None
            for dy in (-1, 0, 1):
                for dx in (-1, 0, 1):
                    ia = (a + dy) % r
                    ib = (bph + dx) % r
                    sy = (a + dy) // r
                    sx = (bph + dx) // r
                    t = _mm(w9[3 * (dy + 1) + (dx + 1)],
                            _sh(zb[(ia, ib)], sy, sx, masks))
                    acc = t if acc is None else acc + t
            out[(a, bph)] = acc + b
    return out


def _cls_body(xc_ref, w0, b0, w1, b1, w2, b2, w3, b3, w4, b4, fcw, fcb,
              route_ref):
    h = _lrelu(_mm(xc_ref[...], w0[...]) + b0[...])
    h = _lrelu(_mm(h, w1[...]) + b1[...])
    h = _lrelu(_mm(h, w2[...]) + b2[...])
    h = _lrelu(_mm(h, w3[...]) + b3[...])
    h = _mm(h, w4[...]) + b4[...]
    m = jnp.mean(h.reshape(64, 64, 32), axis=1)
    logits = _mm(m, fcw[...]) + fcb[...]
    l0 = logits[:, 0:1]
    l1 = logits[:, 1:2]
    l2 = logits[:, 2:3]
    a_col = jnp.where((l0 >= l1) & (l0 >= l2), 0,
                      jnp.where(l1 >= l2, 1, 2)).astype(_I32)
    a_row = a_col.reshape(1, 64)

    ir = jax.lax.broadcasted_iota(_I32, (64, 64), 0)
    ic = jax.lax.broadcasted_iota(_I32, (64, 64), 1)
    before = (a_row < a_col) | ((a_row == a_col) & (ic < ir))
    rank = jnp.sum(before.astype(_I32), axis=1, keepdims=True)
    rank_row = rank.reshape(1, 64)
    sel = (rank_row == ir).astype(_I32)
    s_col = jnp.sum(sel * ic, axis=1, keepdims=True)
    s_row = s_col.reshape(1, 64)

    e8 = jax.lax.broadcasted_iota(_I32, (8, 64), 0)
    cnt8 = jnp.sum((a_row == e8).astype(_I32), axis=1, keepdims=True)
    cnt_row = cnt8.reshape(1, 8)
    c0 = cnt_row[:, 0:1]
    c1 = cnt_row[:, 1:2]
    zero = jnp.zeros((1, 1), _I32)
    off_row = jnp.concatenate([zero, c0, c0 + c1], axis=1)
    cnt_p = jnp.pad(cnt_row, ((0, 0), (0, 56)))
    off_p = jnp.pad(off_row, ((0, 0), (0, 61)))
    zrows = jnp.zeros((4, 64), _I32)
    route_ref[...] = jnp.concatenate(
        [a_row, s_row, cnt_p, off_p, zrows], axis=0)


def _classify(x, cls):
    xc = x.reshape(64, 3, 8, 4, 8, 4).transpose(0, 2, 4, 1, 3, 5)
    xc = xc.reshape(4096, 48)
    args = [
        xc,
        cls['c0_w'].reshape(128, 48).T, cls['c0_b'].reshape(1, 128),
        cls['c1_w'].reshape(128, 128).T, cls['c1_b'].reshape(1, 128),
        cls['c2_w'].reshape(128, 128).T, cls['c2_b'].reshape(1, 128),
        cls['c3_w'].reshape(128, 128).T, cls['c3_b'].reshape(1, 128),
        cls['c4_w'].reshape(32, 128).T, cls['c4_b'].reshape(1, 32),
        cls['fc_w'].T, cls['fc_b'].reshape(1, 3),
    ]
    route = pl.pallas_call(
        _cls_body,
        out_shape=jax.ShapeDtypeStruct((8, 64), _I32),
    )(*args)
    return route


def _carn_weights(p, nf):
    q = nf // 4

    def tapify(w):
        return w.transpose(2, 3, 0, 1).reshape(
            9, w.shape[0], w.shape[1]).astype(_BF16)

    out = [tapify(p['entry_w']), p['entry_b'].reshape(nf, 1)]
    for i in range(3):
        gw = p['b%d_gw' % i]
        full = jnp.zeros((nf, nf, 3, 3), _F32)
        for g in range(4):
            full = full.at[g * q:(g + 1) * q, g * q:(g + 1) * q].set(
                gw[g * q:(g + 1) * q])
        out += [tapify(full), p['b%d_gb' % i].reshape(nf, 1),
                p['b%d_pw' % i].reshape(nf, nf).astype(_BF16),
                p['b%d_pb' % i].reshape(nf, 1)]
    perm = jnp.arange(4 * nf).reshape(4, nf)
    perm = (jnp.arange(nf)[None, :] * 4 + jnp.arange(4)[:, None]).reshape(-1)
    for nm in ('up1', 'up2'):
        w9 = tapify(p[nm + '_w'])
        out += [w9[:, perm, :], p[nm + '_b'][perm].reshape(4 * nf, 1)]
    out += [tapify(p['exit_w']), p['exit_b'].reshape(3, 1)]
    return out


def _carn_body(nf, info_ref, x_ref, ew9, eb, b0w9, b0b, b0pw, b0pb,
               b1w9, b1b, b1pw, b1pb, b2w9, b2b, b2pw, b2pb,
               u1w9, u1b, u2w9, u2b, xw9, xb, out_ref):
    i = pl.program_id(0)

    @pl.when(i < info_ref[1])
    def _():
        masks = _make_masks()
        x = x_ref[0]
        h = jnp.maximum(_conv3(x, ew9, eb[...], masks), 0.0)
        for bw9, bb, bpw, bpb in ((b0w9, b0b, b0pw, b0pb),
                                  (b1w9, b1b, b1pw, b1pb),
                                  (b2w9, b2b, b2pw, b2pb)):
            t = jnp.maximum(_conv3(h, bw9, bb[...], masks), 0.0)
            t = _mm(bpw[...], t.astype(_BF16)) + bpb[...]
            h = jnp.maximum(h + t, 0.0)
        u = jnp.maximum(_conv3(h, u1w9, u1b[...], masks), 0.0)
        z = {(a, b): u[(2 * a + b) * nf:(2 * a + b + 1) * nf]
             for a in range(2) for b in range(2)}
        v = _phase_conv(z, u2w9, u2b[...], 2, masks)
        p16 = {}
        for a in range(2):
            for b in range(2):
                vp = jnp.maximum(v[(a, b)], 0.0)
                for e in range(2):
                    for f in range(2):
                        k = 2 * e + f
                        p16[(2 * a + e, 2 * b + f)] = (
                            vp[k * nf:(k + 1) * nf])
        y = _phase_conv(p16, xw9, xb[...], 4, masks)
        for a in range(4):
            for b in range(4):
                out_ref[0, 4 * a + b] = y[(a, b)]


def _run_expert(xf, info, wlist, nf):
    def in_map(i, info_ref):
        start = info_ref[0]
        count = info_ref[1]
        idx = jnp.minimum(
            start + jnp.minimum(i, jnp.maximum(count - 1, 0)), 63)
        return (info_ref[2 + idx], 0, 0)

    def out_map(i, info_ref):
        start = info_ref[0]
        count = info_ref[1]
        idx = jnp.minimum(
            start + jnp.minimum(i, jnp.maximum(count - 1, 0)), 63)
        slot = jnp.where(i < count, info_ref[2 + idx], 64)
        return (slot, 0, 0, 0)

    in_specs = [pl.BlockSpec((1, 3, 1024), in_map)]
    for w in wlist:
        in_specs.append(pl.BlockSpec(
            w.shape, functools.partial(
                lambda nd, i, info_ref: (0,) * nd, w.ndim)))

    grid_spec = pltpu.PrefetchScalarGridSpec(
        num_scalar_prefetch=1,
        grid=(64,),
        in_specs=in_specs,
        out_specs=pl.BlockSpec((1, 16, 3, 1024), out_map),
    )
    return pl.pallas_call(
        functools.partial(_carn_body, nf),
        grid_spec=grid_spec,
        out_shape=jax.ShapeDtypeStruct((65, 16, 3, 1024), _F32),
    )(info, xf, *wlist)


def _impl(x, params):
    route = _classify(x, params['cls'])
    assign = route[0]
    sorted_idx = route[1]
    counts = route[2, :3]
    offsets = route[3, :3]

    xf = x.reshape(64, 3, 1024)
    ys = []
    for e, (name, nf) in enumerate((('net1', 36), ('net2', 52), ('net3', 64))):
        info = jnp.concatenate(
            [offsets[e:e + 1], counts[e:e + 1], sorted_idx]).astype(_I32)
        wlist = _carn_weights(params[name], nf)
        ys.append(_run_expert(xf, info, wlist, nf))

    sel = assign.reshape(64, 1, 1, 1)
    buf = jnp.where(sel == 0, ys[0][:64],
                    jnp.where(sel == 1, ys[1][:64], ys[2][:64]))
    out = buf.reshape(64, 4, 4, 3, 32, 32).transpose(
        0, 3, 4, 1, 5, 2).reshape(64, 3, 128, 128)
    return out, counts


_impl_jit = jax.jit(_impl)


def kernel(x, params):
    return _impl_jit(x, params)


# speedup vs baseline: 4.1095x; 1.0002x over previous
import functools

import jax
import jax.numpy as jnp
from jax.experimental import pallas as pl
from jax.experimental.pallas import tpu as pltpu

_F32 = jnp.float32
_I32 = jnp.int32
_BF16 = jnp.bfloat16


def _mm(a, b):
    return jnp.dot(a, b, preferred_element_type=_F32)


def _lrelu(x):
    return jnp.where(x >= 0, x, 0.1 * x)


def _make_masks():
    lane = jax.lax.broadcasted_iota(_I32, (1, 1024), 1)
    h = lane // 32
    w = lane % 32
    masks = {}
    for dy in (-1, 0, 1):
        for dx in (-1, 0, 1):
            if dy == 0 and dx == 0:
                continue
            cond = ((h + dy >= 0) & (h + dy < 32)
                    & (w + dx >= 0) & (w + dx < 32))
            masks[(dy, dx)] = cond
    return masks


def _sh(x, dy, dx, masks):
    if dy == 0 and dx == 0:
        return x
    r = jnp.roll(x, -(32 * dy + dx), axis=1)
    return jnp.where(masks[(dy, dx)], r, 0.0)


def _conv3(x, w9, b, masks):
    xb = x.astype(_BF16)
    acc = None
    for ky in range(3):
        for kx in range(3):
            t = _mm(w9[3 * ky + kx], _sh(xb, ky - 1, kx - 1, masks))
            acc = t if acc is None else acc + t
    return acc + b


def _phase_conv(z, w9, b, r, masks):
    zb = {k: v.astype(_BF16) for k, v in z.items()}
    out = {}
    for a in range(r):
        for bph in range(r):
            acc = None
            for dy in (-1, 0, 1):
                for dx in (-1, 0, 1):
                    ia = (a + dy) % r
                    ib = (bph + dx) % r
                    sy = (a + dy) // r
                    sx = (bph + dx) // r
                    t = _mm(w9[3 * (dy + 1) + (dx + 1)],
                            _sh(zb[(ia, ib)], sy, sx, masks))
                    acc = t if acc is None else acc + t
            out[(a, bph)] = acc + b
    return out


def _cls_body(xc_ref, w0, b0, w1, b1, w2, b2, w3, b3, w4, b4, fcw, fcb,
              route_ref):
    h = _lrelu(_mm(xc_ref[...], w0[...]) + b0[...])
    h = _lrelu(_mm(h, w1[...]) + b1[...])
    h = _lrelu(_mm(h, w2[...]) + b2[...])
    h = _lrelu(_mm(h, w3[...]) + b3[...])
    h = _mm(h, w4[...]) + b4[...]
    m = jnp.mean(h.reshape(64, 64, 32), axis=1)
    logits = _mm(m, fcw[...]) + fcb[...]
    l0 = logits[:, 0:1]
    l1 = logits[:, 1:2]
    l2 = logits[:, 2:3]
    a_col = jnp.where((l0 >= l1) & (l0 >= l2), 0,
                      jnp.where(l1 >= l2, 1, 2)).astype(_I32)
    a_row = a_col.reshape(1, 64)

    ir = jax.lax.broadcasted_iota(_I32, (64, 64), 0)
    ic = jax.lax.broadcasted_iota(_I32, (64, 64), 1)
    before = (a_row < a_col) | ((a_row == a_col) & (ic < ir))
    rank = jnp.sum(before.astype(_I32), axis=1, keepdims=True)
    rank_row = rank.reshape(1, 64)
    sel = (rank_row == ir).astype(_I32)
    s_col = jnp.sum(sel * ic, axis=1, keepdims=True)
    s_row = s_col.reshape(1, 64)

    e8 = jax.lax.broadcasted_iota(_I32, (8, 64), 0)
    cnt8 = jnp.sum((a_row == e8).astype(_I32), axis=1, keepdims=True)
    cnt_row = cnt8.reshape(1, 8)
    c0 = cnt_row[:, 0:1]
    c1 = cnt_row[:, 1:2]
    zero = jnp.zeros((1, 1), _I32)
    off_row = jnp.concatenate([zero, c0, c0 + c1], axis=1)
    cnt_p = jnp.pad(cnt_row, ((0, 0), (0, 56)))
    off_p = jnp.pad(off_row, ((0, 0), (0, 61)))
    zrows = jnp.zeros((4, 64), _I32)
    route_ref[...] = jnp.concatenate(
        [a_row, s_row, cnt_p, off_p, zrows], axis=0)


def _classify(x, cls):
    xc = x.reshape(64, 3, 8, 4, 8, 4).transpose(0, 2, 4, 1, 3, 5)
    xc = xc.reshape(4096, 48)
    args = [
        xc,
        cls['c0_w'].reshape(128, 48).T, cls['c0_b'].reshape(1, 128),
        cls['c1_w'].reshape(128, 128).T, cls['c1_b'].reshape(1, 128),
        cls['c2_w'].reshape(128, 128).T, cls['c2_b'].reshape(1, 128),
        cls['c3_w'].reshape(128, 128).T, cls['c3_b'].reshape(1, 128),
        cls['c4_w'].reshape(32, 128).T, cls['c4_b'].reshape(1, 32),
        cls['fc_w'].T, cls['fc_b'].reshape(1, 3),
    ]
    route = pl.pallas_call(
        _cls_body,
        out_shape=jax.ShapeDtypeStruct((8, 64), _I32),
    )(*args)
    return route


def _carn_weights(p, nf):
    q = nf // 4

    def tapify(w):
        return w.transpose(2, 3, 0, 1).reshape(
            9, w.shape[0], w.shape[1]).astype(_BF16)

    out = [tapify(p['entry_w']), p['entry_b'].reshape(nf, 1)]
    for i in range(3):
        gw = p['b%d_gw' % i]
        full = jnp.zeros((nf, nf, 3, 3), _F32)
        for g in range(4):
            full = full.at[g * q:(g + 1) * q, g * q:(g + 1) * q].set(
                gw[g * q:(g + 1) * q])
        out += [tapify(full), p['b%d_gb' % i].reshape(nf, 1),
                p['b%d_pw' % i].reshape(nf, nf).astype(_BF16),
                p['b%d_pb' % i].reshape(nf, 1)]
    perm = (jnp.arange(nf)[None, :] * 4 + jnp.arange(4)[:, None]).reshape(-1)
    for nm in ('up1', 'up2'):
        w9 = tapify(p[nm + '_w'])
        out += [w9[:, perm, :], p[nm + '_b'][perm].reshape(4 * nf, 1)]
    out += [tapify(p['exit_w']), p['exit_b'].reshape(3, 1)]
    return out


def _carn_body(nf, info_ref, x_ref, ew9, eb, b0w9, b0b, b0pw, b0pb,
               b1w9, b1b, b1pw, b1pb, b2w9, b2b, b2pw, b2pb,
               u1w9, u1b, u2w9, u2b, xw9, xb, out_ref):
    i = pl.program_id(0)

    @pl.when(i < info_ref[1])
    def _():
        masks = _make_masks()
        x = x_ref[0]
        h = jnp.maximum(_conv3(x, ew9, eb[...], masks), 0.0)
        for bw9, bb, bpw, bpb in ((b0w9, b0b, b0pw, b0pb),
                                  (b1w9, b1b, b1pw, b1pb),
                                  (b2w9, b2b, b2pw, b2pb)):
            t = jnp.maximum(_conv3(h, bw9, bb[...], masks), 0.0)
            t = _mm(bpw[...], t.astype(_BF16)) + bpb[...]
            h = jnp.maximum(h + t, 0.0)
        u = jnp.maximum(_conv3(h, u1w9, u1b[...], masks), 0.0)
        z = {(a, b): u[(2 * a + b) * nf:(2 * a + b + 1) * nf]
             for a in range(2) for b in range(2)}
        v = _phase_conv(z, u2w9, u2b[...], 2, masks)
        p16 = {}
        for a in range(2):
            for b in range(2):
                vp = jnp.maximum(v[(a, b)], 0.0)
                for e in range(2):
                    for f in range(2):
                        k = 2 * e + f
                        p16[(2 * a + e, 2 * b + f)] = (
                            vp[k * nf:(k + 1) * nf])
        y = _phase_conv(p16, xw9, xb[...], 4, masks)
        for a in range(4):
            for b in range(4):
                out_ref[0, 4 * a + b] = y[(a, b)]


def _run_expert(xf, info, wlist, nf):
    def in_map(i, info_ref):
        start = info_ref[0]
        count = info_ref[1]
        idx = jnp.minimum(
            start + jnp.minimum(i, jnp.maximum(count - 1, 0)), 63)
        return (info_ref[2 + idx], 0, 0)

    def out_map(i, info_ref):
        start = info_ref[0]
        count = info_ref[1]
        idx = jnp.minimum(
            start + jnp.minimum(i, jnp.maximum(count - 1, 0)), 63)
        slot = jnp.where(i < count, info_ref[2 + idx], 64)
        return (slot, 0, 0, 0)

    in_specs = [pl.BlockSpec((1, 3, 1024), in_map)]
    for w in wlist:
        in_specs.append(pl.BlockSpec(
            w.shape, functools.partial(
                lambda nd, i, info_ref: (0,) * nd, w.ndim)))

    grid_spec = pltpu.PrefetchScalarGridSpec(
        num_scalar_prefetch=1,
        grid=(64,),
        in_specs=in_specs,
        out_specs=pl.BlockSpec((1, 16, 3, 1024), out_map),
    )
    return pl.pallas_call(
        functools.partial(_carn_body, nf),
        grid_spec=grid_spec,
        out_shape=jax.ShapeDtypeStruct((65, 16, 3, 1024), _F32),
    )(info, xf, *wlist)


def _impl(x, params):
    route = _classify(x, params['cls'])
    assign = route[0]
    sorted_idx = route[1]
    counts = route[2, :3]
    offsets = route[3, :3]

    xf = x.reshape(64, 3, 1024)
    ys = []
    for e, (name, nf) in enumerate((('net1', 36), ('net2', 52), ('net3', 64))):
        info = jnp.concatenate(
            [offsets[e:e + 1], counts[e:e + 1], sorted_idx]).astype(_I32)
        wlist = _carn_weights(params[name], nf)
        ys.append(_run_expert(xf, info, wlist, nf))

    sel = assign.reshape(64, 1, 1, 1)
    buf = jnp.where(sel == 0, ys[0][:64],
                    jnp.where(sel == 1, ys[1][:64], ys[2][:64]))
    out = buf.reshape(64, 4, 4, 3, 32, 32).transpose(
        0, 3, 4, 1, 5, 2).reshape(64, 3, 128, 128)
    return out, counts


_impl_jit = jax.jit(_impl)


def kernel(x, params):
    return _impl_jit(x, params)
